# Initial kernel scaffold; baseline (speedup 1.0000x reference)
import jax
import jax.numpy as jnp
from jax.experimental import pallas as pl


def kernel(des, tweet, num_prop, cat_prop, new_feature, edge_index, edge_type, W_des, b_des, W_tweet, b_tweet, W_num, b_num, W_cat, b_cat, W_new, b_new, W_in, b_in, W_rel, W_root, b_rgcn, W_out1, b_out1, W_out2, b_out2):
    raise NotImplementedError("write your pallas kernel here")


# speedup vs baseline: 2.5214x; 2.5214x over previous
import functools

import jax
import jax.numpy as jnp
from jax import lax
from jax.experimental import pallas as pl
from jax.experimental.pallas import tpu as pltpu
from jax.experimental.pallas import tpu_sc as plsc

_N = 10000
_E = 320000
_F = 128
_C = 80
_RB = 1000
_KP = 1664
_NS = 16
_EPT = _E // _NS
_CHUNKS = _EPT // _C
_ZR = 640
_AP = _NS * _ZR
_DUMP = _N


def _leaky(v):
    return jnp.where(v > 0, v, 0.01 * v)


def _rb(v):
    return v.astype(jnp.bfloat16).astype(jnp.float32)


def _bdot(a, b):
    return jnp.dot(_rb(a), _rb(b), preferred_element_type=jnp.float32,
                   precision=lax.Precision.HIGHEST)


def _hdot(a, b):
    return jnp.dot(a, _rb(b), preferred_element_type=jnp.float32,
                   precision=lax.Precision.HIGHEST)


def _enc_body(xa, wa, ba, wi, bi, xo):
    h = _leaky(_bdot(xa[...], wa[...]) + ba[...])
    xo[...] = _rb(_leaky(_bdot(h, wi[...]) + bi[...]))


def _encode(xall, wall, ball, w_in, b_in):
    return pl.pallas_call(
        _enc_body,
        grid=(_N // _RB,),
        in_specs=[
            pl.BlockSpec((_RB, _KP), lambda i: (i, 0)),
            pl.BlockSpec((_KP, _F), lambda i: (0, 0)),
            pl.BlockSpec((1, _F), lambda i: (0, 0)),
            pl.BlockSpec((_F, _F), lambda i: (0, 0)),
            pl.BlockSpec((1, _F), lambda i: (0, 0)),
        ],
        out_specs=pl.BlockSpec((_RB, _F), lambda i: (i, 0)),
        out_shape=jax.ShapeDtypeStruct((_N, _F), jnp.float32),
    )(xall, wall, ball, w_in, b_in)


def _combine(x, a0, a1, c0, c1, wroot, b, w0, w1):
    out = _bdot(x[...], wroot[...]) + b[...]
    out = out + _hdot(a0[...], w0[...]) / jnp.clip(c0[...][:, :1], 1.0)
    out = out + _hdot(a1[...], w1[...]) / jnp.clip(c1[...][:, :1], 1.0)
    return out


def _l1_body(x, a0, a1, c0, c1, wroot, b, w0, w1, ho):
    ho[...] = _rb(_combine(x, a0, a1, c0, c1, wroot, b, w0, w1))


def _l2_body(x, a0, a1, c0, c1, wroot, b, w0, w1, wo1, bo1, wo2, bo2, yo):
    h = _combine(x, a0, a1, c0, c1, wroot, b, w0, w1)
    z = _leaky(_bdot(h, wo1[...]) + bo1[...])
    yo[...] = _bdot(z, wo2[...]) + bo2[...]


_ROWB = lambda i: (i, 0)
_FIX = lambda i: (0, 0)

_COMBINE_IN_SPECS = [
    pl.BlockSpec((_RB, _F), _ROWB),
    pl.BlockSpec((_RB, _F), _ROWB),
    pl.BlockSpec((_RB, _F), _ROWB),
    pl.BlockSpec((_RB, 16), _ROWB),
    pl.BlockSpec((_RB, 16), _ROWB),
    pl.BlockSpec((_F, _F), _FIX),
    pl.BlockSpec((1, _F), _FIX),
    pl.BlockSpec((_F, _F), _FIX),
    pl.BlockSpec((_F, _F), _FIX),
]


def _combine1(*args):
    return pl.pallas_call(
        _l1_body,
        grid=(_N // _RB,),
        in_specs=list(_COMBINE_IN_SPECS),
        out_specs=pl.BlockSpec((_RB, _F), _ROWB),
        out_shape=jax.ShapeDtypeStruct((_N, _F), jnp.float32),
    )(*args)


def _combine2(*args):
    return pl.pallas_call(
        _l2_body,
        grid=(_N // _RB,),
        in_specs=list(_COMBINE_IN_SPECS) + [
            pl.BlockSpec((_F, _F), _FIX),
            pl.BlockSpec((1, _F), _FIX),
            pl.BlockSpec((_F, _F), _FIX),
            pl.BlockSpec((1, _F), _FIX),
        ],
        out_specs=pl.BlockSpec((_RB, _F), _ROWB),
        out_shape=jax.ShapeDtypeStruct((_N, _F), jnp.float32),
    )(*args)


def _make_sc_agg(with_counts):
    mesh = plsc.VectorSubcoreMesh(core_axis_name="c", subcore_axis_name="s")
    out_type = [
        jax.ShapeDtypeStruct((2, _AP, _F), jnp.float32),
    ]
    scratch = [
        pltpu.VMEM((_C,), jnp.int32),
        pltpu.VMEM((_C,), jnp.int32),
        pltpu.VMEM((_C,), jnp.int32),
        pltpu.VMEM((_C, _F), jnp.float32),
        pltpu.VMEM_SHARED((_AP, _F), jnp.float32),
        pltpu.SemaphoreType.DMA,
    ]
    if with_counts:
        out_type += [
            jax.ShapeDtypeStruct((2, _AP, _F), jnp.float32),
        ]
        scratch += [
            pltpu.VMEM((_C, _F), jnp.float32),
        ]

    @functools.partial(pl.kernel, mesh=mesh, out_type=out_type,
                       scratch_types=scratch)
    def sc_agg(*refs):
        if with_counts:
            (x, srch, dsth, typh, onesh, zrowh,
             agg, cnt,
             src_v, dst_v, typ_v, rows_v, agg_sh, sem, ones_v) = refs
        else:
            (x, srch, dsth, typh, zrowh,
             agg,
             src_v, dst_v, typ_v, rows_v, agg_sh, sem) = refs

        cid = lax.axis_index("c")
        sid = lax.axis_index("s")
        r0 = sid * _ZR

        pltpu.sync_copy(zrowh, agg_sh.at[pl.ds(r0, _ZR)])
        if with_counts:
            pltpu.sync_copy(onesh, ones_v)
        plsc.subcore_barrier()

        ebase = sid * _EPT

        def mask_dst(sl):
            mine = typ_v[sl] == cid
            dst_v[sl] = jnp.where(mine, dst_v[sl], _DUMP)

        def chunk(g, carry):
            off = ebase + g * _C
            pltpu.sync_copy(srch.at[pl.ds(off, _C)], src_v)
            pltpu.sync_copy(dsth.at[pl.ds(off, _C)], dst_v)
            pltpu.sync_copy(typh.at[pl.ds(off, _C)], typ_v)
            for j in range(_C // 16):
                mask_dst(pl.ds(j * 16, 16))
            pltpu.async_copy(x.at[src_v], rows_v, sem).wait()
            pltpu.sync_copy(rows_v, agg_sh.at[dst_v], add=True)
            return carry

        lax.fori_loop(0, _CHUNKS, chunk, 0)
        plsc.subcore_barrier()

        pltpu.sync_copy(agg_sh.at[pl.ds(r0, _ZR)],
                        agg.at[cid, pl.ds(r0, _ZR)])

        if with_counts:
            plsc.subcore_barrier()
            pltpu.sync_copy(zrowh, agg_sh.at[pl.ds(r0, _ZR)])
            plsc.subcore_barrier()

            def cchunk(g, carry):
                off = ebase + g * _C
                pltpu.sync_copy(dsth.at[pl.ds(off, _C)], dst_v)
                pltpu.sync_copy(typh.at[pl.ds(off, _C)], typ_v)
                for j in range(_C // 16):
                    mask_dst(pl.ds(j * 16, 16))
                pltpu.sync_copy(ones_v, agg_sh.at[dst_v], add=True)
                return carry

            lax.fori_loop(0, _CHUNKS, cchunk, 0)
            plsc.subcore_barrier()
            pltpu.sync_copy(agg_sh.at[pl.ds(r0, _ZR)],
                            cnt.at[cid, pl.ds(r0, _ZR)])

    return sc_agg


_sc_cache = {}


def _get_sc_agg(with_counts):
    if with_counts not in _sc_cache:
        _sc_cache[with_counts] = _make_sc_agg(with_counts)
    return _sc_cache[with_counts]


def kernel(des, tweet, num_prop, cat_prop, new_feature, edge_index, edge_type,
           W_des, b_des, W_tweet, b_tweet, W_num, b_num, W_cat, b_cat,
           W_new, b_new, W_in, b_in, W_rel, W_root, b_rgcn,
           W_out1, b_out1, W_out2, b_out2):
    f32 = jnp.float32

    xall = jnp.concatenate(
        [des, tweet, num_prop, cat_prop, new_feature], axis=1)
    xall = jnp.pad(xall, ((0, 0), (0, _KP - 1556)))
    wall = jnp.zeros((_KP, _F), f32)
    wall = wall.at[0:768, 0:25].set(W_des)
    wall = wall.at[768:1536, 25:53].set(W_tweet)
    wall = wall.at[1536:1543, 53:78].set(W_num)
    wall = wall.at[1543:1554, 78:103].set(W_cat)
    wall = wall.at[1554:1556, 103:128].set(W_new)
    ball = jnp.concatenate(
        [b_des, b_tweet, b_num, b_cat, b_new]).reshape(1, _F)

    x = _encode(xall, wall, ball, W_in, b_in.reshape(1, _F))

    src = edge_index[0]
    dst = edge_index[1]
    ones = jnp.ones((_C, _F), f32)
    zrow = jnp.zeros((_ZR, _F), f32)

    agg, cnt = _get_sc_agg(True)(x, src, dst, edge_type, ones, zrow)
    c0 = cnt[0, :_N, :16]
    c1 = cnt[1, :_N, :16]

    brg = b_rgcn.reshape(1, _F)
    h = _combine1(x, agg[0, :_N], agg[1, :_N], c0, c1,
                  W_root, brg, W_rel[0], W_rel[1])

    aggb, = _get_sc_agg(False)(h, src, dst, edge_type, zrow)

    w2p = jnp.zeros((_F, _F), f32).at[:, :2].set(W_out2)
    b2p = jnp.zeros((1, _F), f32).at[0, :2].set(b_out2)
    ypad = _combine2(h, aggb[0, :_N], aggb[1, :_N], c0, c1,
                     W_root, brg, W_rel[0], W_rel[1],
                     W_out1, b_out1.reshape(1, _F), w2p, b2p)
    return ypad[:, :2]


# speedup vs baseline: 4.5004x; 1.7848x over previous
import functools

import jax
import jax.numpy as jnp
from jax import lax
from jax.experimental import pallas as pl
from jax.experimental.pallas import tpu as pltpu
from jax.experimental.pallas import tpu_sc as plsc

_N = 10000
_E = 320000
_F = 128
_C = 80
_RB = 1000
_KP = 1664
_NS = 16
_EPT = _E // _NS
_CHUNKS = _EPT // _C
_K = 2
_ROUNDS = _CHUNKS // _K
_ZR = 640
_AP = _NS * _ZR
_DUMP = _N


def _leaky(v):
    return jnp.where(v > 0, v, 0.01 * v)


def _rb(v):
    return v.astype(jnp.bfloat16).astype(jnp.float32)


def _bdot(a, b):
    return jnp.dot(_rb(a), _rb(b), preferred_element_type=jnp.float32,
                   precision=lax.Precision.HIGHEST)


def _hdot(a, b):
    return jnp.dot(a, _rb(b), preferred_element_type=jnp.float32,
                   precision=lax.Precision.HIGHEST)


def _enc_body(xa, wa, ba, wi, bi, xo):
    h = _leaky(_bdot(xa[...], wa[...]) + ba[...])
    xo[...] = _rb(_leaky(_bdot(h, wi[...]) + bi[...]))


def _encode(xall, wall, ball, w_in, b_in):
    return pl.pallas_call(
        _enc_body,
        grid=(_N // _RB,),
        in_specs=[
            pl.BlockSpec((_RB, _KP), lambda i: (i, 0)),
            pl.BlockSpec((_KP, _F), lambda i: (0, 0)),
            pl.BlockSpec((1, _F), lambda i: (0, 0)),
            pl.BlockSpec((_F, _F), lambda i: (0, 0)),
            pl.BlockSpec((1, _F), lambda i: (0, 0)),
        ],
        out_specs=pl.BlockSpec((_RB, _F), lambda i: (i, 0)),
        out_shape=jax.ShapeDtypeStruct((_N, _F), jnp.float32),
    )(xall, wall, ball, w_in, b_in)


def _combine(x, a0, a1, c0, c1, wroot, b, w0, w1):
    out = _bdot(x[...], wroot[...]) + b[...]
    out = out + _hdot(a0[...], w0[...]) / jnp.clip(c0[...][:, :1], 1.0)
    out = out + _hdot(a1[...], w1[...]) / jnp.clip(c1[...][:, :1], 1.0)
    return out


def _l1_body(x, a0, a1, c0, c1, wroot, b, w0, w1, ho):
    ho[...] = _rb(_combine(x, a0, a1, c0, c1, wroot, b, w0, w1))


def _l2_body(x, a0, a1, c0, c1, wroot, b, w0, w1, wo1, bo1, wo2, bo2, yo):
    h = _combine(x, a0, a1, c0, c1, wroot, b, w0, w1)
    z = _leaky(_bdot(h, wo1[...]) + bo1[...])
    yo[...] = _bdot(z, wo2[...]) + bo2[...]


_ROWB = lambda i: (i, 0)
_FIX = lambda i: (0, 0)

_COMBINE_IN_SPECS = [
    pl.BlockSpec((_RB, _F), _ROWB),
    pl.BlockSpec((_RB, _F), _ROWB),
    pl.BlockSpec((_RB, _F), _ROWB),
    pl.BlockSpec((_RB, 16), _ROWB),
    pl.BlockSpec((_RB, 16), _ROWB),
    pl.BlockSpec((_F, _F), _FIX),
    pl.BlockSpec((1, _F), _FIX),
    pl.BlockSpec((_F, _F), _FIX),
    pl.BlockSpec((_F, _F), _FIX),
]


def _combine1(*args):
    return pl.pallas_call(
        _l1_body,
        grid=(_N // _RB,),
        in_specs=list(_COMBINE_IN_SPECS),
        out_specs=pl.BlockSpec((_RB, _F), _ROWB),
        out_shape=jax.ShapeDtypeStruct((_N, _F), jnp.float32),
    )(*args)


def _combine2(*args):
    return pl.pallas_call(
        _l2_body,
        grid=(_N // _RB,),
        in_specs=list(_COMBINE_IN_SPECS) + [
            pl.BlockSpec((_F, _F), _FIX),
            pl.BlockSpec((1, _F), _FIX),
            pl.BlockSpec((_F, _F), _FIX),
            pl.BlockSpec((1, _F), _FIX),
        ],
        out_specs=pl.BlockSpec((_RB, _F), _ROWB),
        out_shape=jax.ShapeDtypeStruct((_N, _F), jnp.float32),
    )(*args)


def _make_sc_agg(with_counts):
    mesh = plsc.VectorSubcoreMesh(core_axis_name="c", subcore_axis_name="s")
    out_type = [
        jax.ShapeDtypeStruct((2, _AP, _F), jnp.float32),
    ]
    scratch = [
        pltpu.VMEM((_EPT,), jnp.int32),
        pltpu.VMEM_SHARED((_AP, _F), jnp.float32),
    ]
    scratch += [pltpu.VMEM((_C, _F), jnp.float32) for _ in range(_K)]
    scratch += [pltpu.VMEM((_C,), jnp.int32) for _ in range(_K)]
    scratch += [pltpu.VMEM((_C,), jnp.int32) for _ in range(_K)]
    scratch += [pltpu.VMEM((_C,), jnp.int32) for _ in range(_K)]
    scratch += [pltpu.SemaphoreType.DMA for _ in range(3 * _K)]
    if with_counts:
        out_type += [
            jax.ShapeDtypeStruct((2, _AP, _F), jnp.float32),
        ]

    @functools.partial(pl.kernel, mesh=mesh, out_type=out_type,
                       scratch_types=scratch)
    def sc_agg(*refs):
        if with_counts:
            (x, srch, dsth, typh, onesh, zrowh, agg, cnt, *bufs) = refs
        else:
            (x, srch, dsth, typh, zrowh, agg, *bufs) = refs
        srca, agg_sh = bufs[:2]
        rows = bufs[2:2 + _K]
        dtmp = bufs[2 + _K:2 + 2 * _K]
        ttmp = bufs[2 + 2 * _K:2 + 3 * _K]
        dstb = bufs[2 + 3 * _K:2 + 4 * _K]
        isem = bufs[2 + 4 * _K:2 + 5 * _K]
        gsem = bufs[2 + 5 * _K:2 + 6 * _K]
        ssem = bufs[2 + 6 * _K:2 + 7 * _K]

        cid = lax.axis_index("c")
        sid = lax.axis_index("s")
        r0 = sid * _ZR
        ebase = sid * _EPT

        pltpu.sync_copy(srch.at[pl.ds(ebase, _EPT)], srca)
        pltpu.sync_copy(zrowh, agg_sh.at[pl.ds(r0, _ZR)])
        plsc.subcore_barrier()

        def fire_idx(k, off):
            pltpu.async_copy(dsth.at[pl.ds(ebase + off, _C)], dtmp[k],
                             isem[k])
            pltpu.async_copy(typh.at[pl.ds(ebase + off, _C)], ttmp[k],
                             isem[k])

        def wait_idx(k, off):
            pltpu.make_async_copy(dsth.at[pl.ds(ebase + off, _C)], dtmp[k],
                                  isem[k]).wait()
            pltpu.make_async_copy(typh.at[pl.ds(ebase + off, _C)], ttmp[k],
                                  isem[k]).wait()

        def mask_dst(k):
            for j in range(_C // 16):
                sl = pl.ds(j * 16, 16)
                dstb[k][sl] = jnp.where(ttmp[k][sl] == cid, dtmp[k][sl],
                                        _DUMP)

        def wait_scat(k, src_buf):
            pltpu.make_async_copy(src_buf, agg_sh.at[dstb[k]],
                                  ssem[k]).wait()

        def rnd(r, carry):
            for k in range(_K):
                off = (r * _K + k) * _C

                @pl.when(r > 0)
                def _():
                    wait_scat(k, rows[k])
                fire_idx(k, off)
                pltpu.async_copy(x.at[srca.at[pl.ds(off, _C)]],
                                 rows[k], gsem[k])
            for k in range(_K):
                off = (r * _K + k) * _C
                wait_idx(k, off)
                pltpu.make_async_copy(x.at[srca.at[pl.ds(off, _C)]],
                                      rows[k], gsem[k]).wait()
                mask_dst(k)
                pltpu.async_copy(rows[k], agg_sh.at[dstb[k]], ssem[k],
                                 add=True)
            return carry

        lax.fori_loop(0, _ROUNDS, rnd, 0)
        for k in range(_K):
            wait_scat(k, rows[k])
        plsc.subcore_barrier()

        pltpu.sync_copy(agg_sh.at[pl.ds(r0, _ZR)],
                        agg.at[cid, pl.ds(r0, _ZR)])

        if with_counts:
            plsc.subcore_barrier()
            pltpu.sync_copy(zrowh, agg_sh.at[pl.ds(r0, _ZR)])
            pltpu.sync_copy(onesh, rows[0])
            plsc.subcore_barrier()

            def crnd(r, carry):
                for k in range(_K):
                    off = (r * _K + k) * _C

                    @pl.when(r > 0)
                    def _():
                        wait_scat(k, rows[0])
                    fire_idx(k, off)
                for k in range(_K):
                    off = (r * _K + k) * _C
                    wait_idx(k, off)
                    mask_dst(k)
                    pltpu.async_copy(rows[0], agg_sh.at[dstb[k]], ssem[k],
                                     add=True)
                return carry

            lax.fori_loop(0, _ROUNDS, crnd, 0)
            for k in range(_K):
                wait_scat(k, rows[0])
            plsc.subcore_barrier()
            pltpu.sync_copy(agg_sh.at[pl.ds(r0, _ZR)],
                            cnt.at[cid, pl.ds(r0, _ZR)])

    return sc_agg


_sc_cache = {}


def _get_sc_agg(with_counts):
    if with_counts not in _sc_cache:
        _sc_cache[with_counts] = _make_sc_agg(with_counts)
    return _sc_cache[with_counts]


def kernel(des, tweet, num_prop, cat_prop, new_feature, edge_index, edge_type,
           W_des, b_des, W_tweet, b_tweet, W_num, b_num, W_cat, b_cat,
           W_new, b_new, W_in, b_in, W_rel, W_root, b_rgcn,
           W_out1, b_out1, W_out2, b_out2):
    f32 = jnp.float32

    xall = jnp.concatenate(
        [des, tweet, num_prop, cat_prop, new_feature], axis=1)
    xall = jnp.pad(xall, ((0, 0), (0, _KP - 1556)))
    wall = jnp.zeros((_KP, _F), f32)
    wall = wall.at[0:768, 0:25].set(W_des)
    wall = wall.at[768:1536, 25:53].set(W_tweet)
    wall = wall.at[1536:1543, 53:78].set(W_num)
    wall = wall.at[1543:1554, 78:103].set(W_cat)
    wall = wall.at[1554:1556, 103:128].set(W_new)
    ball = jnp.concatenate(
        [b_des, b_tweet, b_num, b_cat, b_new]).reshape(1, _F)

    x = _encode(xall, wall, ball, W_in, b_in.reshape(1, _F))

    src = edge_index[0]
    dst = edge_index[1]
    ones = jnp.ones((_C, _F), f32)
    zrow = jnp.zeros((_ZR, _F), f32)

    agg, cnt = _get_sc_agg(True)(x, src, dst, edge_type, ones, zrow)
    c0 = cnt[0, :_N, :16]
    c1 = cnt[1, :_N, :16]

    brg = b_rgcn.reshape(1, _F)
    h = _combine1(x, agg[0, :_N], agg[1, :_N], c0, c1,
                  W_root, brg, W_rel[0], W_rel[1])

    aggb, = _get_sc_agg(False)(h, src, dst, edge_type, zrow)

    w2p = jnp.zeros((_F, _F), f32).at[:, :2].set(W_out2)
    b2p = jnp.zeros((1, _F), f32).at[0, :2].set(b_out2)
    ypad = _combine2(h, aggb[0, :_N], aggb[1, :_N], c0, c1,
                     W_root, brg, W_rel[0], W_rel[1],
                     W_out1, b_out1.reshape(1, _F), w2p, b2p)
    return ypad[:, :2]


# speedup vs baseline: 4.9647x; 1.1032x over previous
import functools

import jax
import jax.numpy as jnp
from jax import lax
from jax.experimental import pallas as pl
from jax.experimental.pallas import tpu as pltpu
from jax.experimental.pallas import tpu_sc as plsc

_N = 10000
_E = 320000
_F = 128
_C = 80
_RB = 1000
_KP = 1664
_NS = 16
_EPT = _E // _NS
_CHUNKS = _EPT // _C
_K = 2
_ROUNDS = _CHUNKS // _K
_TAIL = _CHUNKS - _K * _ROUNDS
_ZR = 640
_AP = _NS * _ZR
_DUMP = _N


def _leaky(v):
    return jnp.where(v > 0, v, 0.01 * v)


def _rb(v):
    return v.astype(jnp.bfloat16).astype(jnp.float32)


def _bdot(a, b):
    return jnp.dot(_rb(a), _rb(b), preferred_element_type=jnp.float32,
                   precision=lax.Precision.HIGHEST)


def _hdot(a, b):
    return jnp.dot(a, _rb(b), preferred_element_type=jnp.float32,
                   precision=lax.Precision.HIGHEST)


def _enc_body(de, tw, sm, wd, wt, ws, ba, wi, bi, xo):
    h = (_bdot(de[...], wd[...]) + _bdot(tw[...], wt[...])
         + _bdot(sm[...], ws[...]) + ba[...])
    h = _leaky(h)
    xo[...] = _rb(_leaky(_bdot(h, wi[...]) + bi[...]))


def _encode(des, tweet, small, wd, wt, ws, ball, w_in, b_in):
    return pl.pallas_call(
        _enc_body,
        grid=(_N // _RB,),
        in_specs=[
            pl.BlockSpec((_RB, 768), lambda i: (i, 0)),
            pl.BlockSpec((_RB, 768), lambda i: (i, 0)),
            pl.BlockSpec((_RB, _F), lambda i: (i, 0)),
            pl.BlockSpec((768, _F), lambda i: (0, 0)),
            pl.BlockSpec((768, _F), lambda i: (0, 0)),
            pl.BlockSpec((_F, _F), lambda i: (0, 0)),
            pl.BlockSpec((1, _F), lambda i: (0, 0)),
            pl.BlockSpec((_F, _F), lambda i: (0, 0)),
            pl.BlockSpec((1, _F), lambda i: (0, 0)),
        ],
        out_specs=pl.BlockSpec((_RB, _F), lambda i: (i, 0)),
        out_shape=jax.ShapeDtypeStruct((_N, _F), jnp.float32),
    )(des, tweet, small, wd, wt, ws, ball, w_in, b_in)


def _combine(x, a0, a1, c0, c1, wroot, b, w0, w1):
    out = _bdot(x[...], wroot[...]) + b[...]
    out = out + _hdot(a0[...], w0[...]) / jnp.clip(c0[...][:, :1], 1.0)
    out = out + _hdot(a1[...], w1[...]) / jnp.clip(c1[...][:, :1], 1.0)
    return out


def _l1_body(x, a0, a1, c0, c1, wroot, b, w0, w1, ho):
    ho[...] = _rb(_combine(x, a0, a1, c0, c1, wroot, b, w0, w1))


def _l2_body(x, a0, a1, c0, c1, wroot, b, w0, w1, wo1, bo1, wo2, bo2, yo):
    h = _combine(x, a0, a1, c0, c1, wroot, b, w0, w1)
    z = _leaky(_bdot(h, wo1[...]) + bo1[...])
    yo[...] = _bdot(z, wo2[...]) + bo2[...]


_ROWB = lambda i: (i, 0)
_FIX = lambda i: (0, 0)

_COMBINE_IN_SPECS = [
    pl.BlockSpec((_RB, _F), _ROWB),
    pl.BlockSpec((_RB, _F), _ROWB),
    pl.BlockSpec((_RB, _F), _ROWB),
    pl.BlockSpec((_RB, 16), _ROWB),
    pl.BlockSpec((_RB, 16), _ROWB),
    pl.BlockSpec((_F, _F), _FIX),
    pl.BlockSpec((1, _F), _FIX),
    pl.BlockSpec((_F, _F), _FIX),
    pl.BlockSpec((_F, _F), _FIX),
]


def _combine1(*args):
    return pl.pallas_call(
        _l1_body,
        grid=(_N // _RB,),
        in_specs=list(_COMBINE_IN_SPECS),
        out_specs=pl.BlockSpec((_RB, _F), _ROWB),
        out_shape=jax.ShapeDtypeStruct((_N, _F), jnp.float32),
    )(*args)


def _combine2(*args):
    return pl.pallas_call(
        _l2_body,
        grid=(_N // _RB,),
        in_specs=list(_COMBINE_IN_SPECS) + [
            pl.BlockSpec((_F, _F), _FIX),
            pl.BlockSpec((1, _F), _FIX),
            pl.BlockSpec((_F, _F), _FIX),
            pl.BlockSpec((1, _F), _FIX),
        ],
        out_specs=pl.BlockSpec((_RB, _F), _ROWB),
        out_shape=jax.ShapeDtypeStruct((_N, _F), jnp.float32),
    )(*args)


def _make_sc_agg(with_counts):
    mesh = plsc.VectorSubcoreMesh(core_axis_name="c", subcore_axis_name="s")
    out_type = [
        jax.ShapeDtypeStruct((2, _AP, _F), jnp.float32),
    ]
    scratch = [
        pltpu.VMEM((_EPT,), jnp.int32),
        pltpu.VMEM_SHARED((_AP, _F), jnp.float32),
    ]
    scratch += [pltpu.VMEM((_C, _F), jnp.float32) for _ in range(_K)]
    scratch += [pltpu.VMEM((_C,), jnp.int32) for _ in range(_K)]
    scratch += [pltpu.VMEM((_C,), jnp.int32) for _ in range(_K)]
    scratch += [pltpu.VMEM((_C,), jnp.int32) for _ in range(_K)]
    scratch += [pltpu.SemaphoreType.DMA for _ in range(3 * _K)]
    if with_counts:
        out_type += [
            jax.ShapeDtypeStruct((2, _AP, _F), jnp.float32),
        ]

    @functools.partial(pl.kernel, mesh=mesh, out_type=out_type,
                       scratch_types=scratch)
    def sc_agg(*refs):
        if with_counts:
            (x, srch, dsth, typh, onesh, zrowh, agg, cnt, *bufs) = refs
        else:
            (x, srch, dsth, typh, zrowh, agg, *bufs) = refs
        srca, agg_sh = bufs[:2]
        rows = bufs[2:2 + _K]
        dtmp = bufs[2 + _K:2 + 2 * _K]
        ttmp = bufs[2 + 2 * _K:2 + 3 * _K]
        dstb = bufs[2 + 3 * _K:2 + 4 * _K]
        isem = bufs[2 + 4 * _K:2 + 5 * _K]
        gsem = bufs[2 + 5 * _K:2 + 6 * _K]
        ssem = bufs[2 + 6 * _K:2 + 7 * _K]

        cid = lax.axis_index("c")
        sid = lax.axis_index("s")
        r0 = sid * _ZR
        ebase = sid * _EPT

        pltpu.sync_copy(srch.at[pl.ds(ebase, _EPT)], srca)
        pltpu.sync_copy(zrowh, agg_sh.at[pl.ds(r0, _ZR)])
        plsc.subcore_barrier()

        def fire_idx(k, off):
            pltpu.async_copy(dsth.at[pl.ds(ebase + off, _C)], dtmp[k],
                             isem[k])
            pltpu.async_copy(typh.at[pl.ds(ebase + off, _C)], ttmp[k],
                             isem[k])

        def wait_idx(k, off):
            pltpu.make_async_copy(dsth.at[pl.ds(ebase + off, _C)], dtmp[k],
                                  isem[k]).wait()
            pltpu.make_async_copy(typh.at[pl.ds(ebase + off, _C)], ttmp[k],
                                  isem[k]).wait()

        def mask_dst(k):
            for j in range(_C // 16):
                sl = pl.ds(j * 16, 16)
                dstb[k][sl] = jnp.where(ttmp[k][sl] == cid, dtmp[k][sl],
                                        _DUMP)

        def wait_scat(k, src_buf):
            pltpu.make_async_copy(src_buf, agg_sh.at[dstb[k]],
                                  ssem[k]).wait()

        def gphase1(cbase, ks, guard):
            for k in ks:
                off = (cbase + k) * _C
                if guard:
                    @pl.when(guard())
                    def _():
                        wait_scat(k, rows[k])
                else:
                    wait_scat(k, rows[k])
                fire_idx(k, off)
                pltpu.async_copy(x.at[srca.at[pl.ds(off, _C)]],
                                 rows[k], gsem[k])

        def gphase2(cbase, ks):
            for k in ks:
                off = (cbase + k) * _C
                wait_idx(k, off)
                pltpu.make_async_copy(x.at[srca.at[pl.ds(off, _C)]],
                                      rows[k], gsem[k]).wait()
                mask_dst(k)
                pltpu.async_copy(rows[k], agg_sh.at[dstb[k]], ssem[k],
                                 add=True)

        def rnd(r, carry):
            gphase1(r * _K, range(_K), lambda: r > 0)
            gphase2(r * _K, range(_K))
            return carry

        lax.fori_loop(0, _ROUNDS, rnd, 0)
        if _TAIL:
            gphase1(_ROUNDS * _K, range(_TAIL), None)
            gphase2(_ROUNDS * _K, range(_TAIL))
        for k in range(_K):
            wait_scat(k, rows[k])
        plsc.subcore_barrier()

        pltpu.sync_copy(agg_sh.at[pl.ds(r0, _ZR)],
                        agg.at[cid, pl.ds(r0, _ZR)])

        if with_counts:
            plsc.subcore_barrier()
            pltpu.sync_copy(zrowh, agg_sh.at[pl.ds(r0, _ZR)])
            pltpu.sync_copy(onesh, rows[0])
            plsc.subcore_barrier()

            def cphase1(cbase, ks, guard):
                for k in ks:
                    off = (cbase + k) * _C
                    if guard:
                        @pl.when(guard())
                        def _():
                            wait_scat(k, rows[0])
                    else:
                        wait_scat(k, rows[0])
                    fire_idx(k, off)

            def cphase2(cbase, ks):
                for k in ks:
                    off = (cbase + k) * _C
                    wait_idx(k, off)
                    mask_dst(k)
                    pltpu.async_copy(rows[0], agg_sh.at[dstb[k]], ssem[k],
                                     add=True)

            def crnd(r, carry):
                cphase1(r * _K, range(_K), lambda: r > 0)
                cphase2(r * _K, range(_K))
                return carry

            lax.fori_loop(0, _ROUNDS, crnd, 0)
            if _TAIL:
                cphase1(_ROUNDS * _K, range(_TAIL), None)
                cphase2(_ROUNDS * _K, range(_TAIL))
            for k in range(_K):
                wait_scat(k, rows[0])
            plsc.subcore_barrier()
            pltpu.sync_copy(agg_sh.at[pl.ds(r0, _ZR)],
                            cnt.at[cid, pl.ds(r0, _ZR)])

    return sc_agg


_sc_cache = {}


def _get_sc_agg(with_counts):
    if with_counts not in _sc_cache:
        _sc_cache[with_counts] = _make_sc_agg(with_counts)
    return _sc_cache[with_counts]


def kernel(des, tweet, num_prop, cat_prop, new_feature, edge_index, edge_type,
           W_des, b_des, W_tweet, b_tweet, W_num, b_num, W_cat, b_cat,
           W_new, b_new, W_in, b_in, W_rel, W_root, b_rgcn,
           W_out1, b_out1, W_out2, b_out2):
    f32 = jnp.float32

    wd = jnp.zeros((768, _F), f32).at[:, 0:25].set(W_des)
    wt = jnp.zeros((768, _F), f32).at[:, 25:53].set(W_tweet)
    ws = jnp.zeros((_F, _F), f32)
    ws = ws.at[0:7, 53:78].set(W_num)
    ws = ws.at[7:18, 78:103].set(W_cat)
    ws = ws.at[18:20, 103:128].set(W_new)
    small = jnp.pad(
        jnp.concatenate([num_prop, cat_prop, new_feature], axis=1),
        ((0, 0), (0, _F - 20)))
    ball = jnp.concatenate(
        [b_des, b_tweet, b_num, b_cat, b_new]).reshape(1, _F)

    x = _encode(des, tweet, small, wd, wt, ws, ball, W_in,
                b_in.reshape(1, _F))

    src = edge_index[0]
    dst = edge_index[1]
    ones = jnp.ones((_C, _F), f32)
    zrow = jnp.zeros((_ZR, _F), f32)

    agg, cnt = _get_sc_agg(True)(x, src, dst, edge_type, ones, zrow)
    c0 = cnt[0, :_N, :16]
    c1 = cnt[1, :_N, :16]

    brg = b_rgcn.reshape(1, _F)
    h = _combine1(x, agg[0, :_N], agg[1, :_N], c0, c1,
                  W_root, brg, W_rel[0], W_rel[1])

    aggb, = _get_sc_agg(False)(h, src, dst, edge_type, zrow)

    w2p = jnp.zeros((_F, _F), f32).at[:, :2].set(W_out2)
    b2p = jnp.zeros((1, _F), f32).at[0, :2].set(b_out2)
    ypad = _combine2(h, aggb[0, :_N], aggb[1, :_N], c0, c1,
                     W_root, brg, W_rel[0], W_rel[1],
                     W_out1, b_out1.reshape(1, _F), w2p, b2p)
    return ypad[:, :2]
